# baseline (speedup 1.0000x reference)
import functools

import jax
import jax.numpy as jnp
from jax import lax
from jax.experimental import pallas as pl
from jax.experimental.pallas import tpu as pltpu
from jax.experimental.pallas import tpu_sc as plsc

_VOCAB = 100000
_ROWS = 16
_COLS = 4096
_LANES = 16
_CHUNKS = _COLS // _LANES


def _tokenizer_body(inp_hbm, packed_hbm, len_hbm, x_v, out_v, len_v):
    c = lax.axis_index("c")
    s = lax.axis_index("s")

    @pl.when(s < 8)
    def _():
        r = c * 8 + s
        pltpu.sync_copy(inp_hbm.at[r], x_v)

        one = jnp.full((_LANES,), 1, jnp.int32)
        zero = jnp.full((_LANES,), 0, jnp.int32)
        neg1 = jnp.full((_LANES,), -1, jnp.int32)

        @plsc.parallel_loop(0, _CHUNKS, unroll=8)
        def _fill(i):
            out_v[pl.ds(i * _LANES, _LANES)] = neg1

        @plsc.parallel_loop(0, _CHUNKS, unroll=4, carry=zero)
        def total_v(i, off_v):
            v = x_v[pl.ds(i * _LANES, _LANES)]
            m = v < _VOCAB
            mi = jnp.where(m, one, zero)
            incl = plsc.cumsum(mi)
            idx = off_v + (incl - mi)
            plsc.store_scatter(out_v, [idx], v, mask=m)
            return off_v + incl[15]

        pltpu.sync_copy(out_v, packed_hbm.at[r])
        len_v[...] = total_v
        pltpu.sync_copy(len_v.at[pl.ds(0, 8)], len_hbm.at[pl.ds(r * 8, 8)])


@jax.jit
def kernel(inputs):
    mesh = plsc.VectorSubcoreMesh(core_axis_name="c", subcore_axis_name="s")
    call = pl.kernel(
        _tokenizer_body,
        mesh=mesh,
        compiler_params=pltpu.CompilerParams(needs_layout_passes=False),
        out_type=[
            jax.ShapeDtypeStruct((_ROWS, _COLS), jnp.int32),
            jax.ShapeDtypeStruct((_ROWS * 8,), jnp.int32),
        ],
        scratch_types=[
            pltpu.VMEM((_COLS,), jnp.int32),
            pltpu.VMEM((_COLS,), jnp.int32),
            pltpu.VMEM((_LANES,), jnp.int32),
        ],
    )
    packed, len8 = call(inputs)
    return packed, len8[::8]


# speedup vs baseline: 1.0033x; 1.0033x over previous
import jax
import jax.numpy as jnp
from jax import lax
from jax.experimental import pallas as pl
from jax.experimental.pallas import tpu as pltpu
from jax.experimental.pallas import tpu_sc as plsc

_VOCAB = 100000
_ROWS = 16
_COLS = 4096
_LANES = 16
_CHUNKS = _COLS // _LANES


def _tokenizer_body(inp_hbm, packed_hbm, len_hbm, x_v, out_v, len_v):
    c = lax.axis_index("c")
    s = lax.axis_index("s")

    @pl.when(s < 8)
    def _():
        r = c * 8 + s
        pltpu.sync_copy(inp_hbm.at[r], x_v)

        one = jnp.full((_LANES,), 1, jnp.int32)
        zero = jnp.full((_LANES,), 0, jnp.int32)
        neg1 = jnp.full((_LANES,), -1, jnp.int32)

        @plsc.parallel_loop(0, _CHUNKS, unroll=8)
        def _fill(i):
            out_v[pl.ds(i * _LANES, _LANES)] = neg1

        @plsc.parallel_loop(0, _CHUNKS, unroll=4, carry=zero)
        def total_v(i, off_v):
            v = x_v[pl.ds(i * _LANES, _LANES)]
            m = v < _VOCAB
            mi = jnp.where(m, one, zero)
            incl = plsc.cumsum(mi)
            idx = off_v + (incl - mi)
            plsc.store_scatter(out_v, [idx], v, mask=m)
            return off_v + incl[15]

        pltpu.sync_copy(out_v, packed_hbm.at[r])
        len_v[...] = total_v
        pltpu.sync_copy(len_v.at[pl.ds(0, 8)], len_hbm.at[pl.ds(r * 8, 8)])


@jax.jit
def kernel(inputs):
    mesh = plsc.VectorSubcoreMesh(core_axis_name="c", subcore_axis_name="s")
    call = pl.kernel(
        _tokenizer_body,
        mesh=mesh,
        compiler_params=pltpu.CompilerParams(needs_layout_passes=False),
        out_type=[
            jax.ShapeDtypeStruct((_ROWS, _COLS), jnp.int32),
            jax.ShapeDtypeStruct((_ROWS * 8,), jnp.int32),
        ],
        scratch_types=[
            pltpu.VMEM((_COLS,), jnp.int32),
            pltpu.VMEM((_COLS,), jnp.int32),
            pltpu.VMEM((_LANES,), jnp.int32),
        ],
    )
    packed, len8 = call(inputs)
    return packed, len8[::8]
